# Initial kernel scaffold; baseline (speedup 1.0000x reference)
import jax
import jax.numpy as jnp
from jax.experimental import pallas as pl


def kernel(img_lr, h_weight, d_weight, b_weight, v_weight):
    raise NotImplementedError("write your pallas kernel here")


# speedup vs baseline: 717.6948x; 717.6948x over previous
import jax
import jax.numpy as jnp
from jax import lax
from jax.experimental import pallas as pl
from jax.experimental.pallas import tpu as pltpu
from jax.experimental.pallas import tpu_sc as plsc

_L = 17
_N = 384
_PLANES = 6
_NW = 32
_RPW = _N // _NW
_WIN = _RPW + 4
_TAB = 4920
_GROUPS = _N // 16
_OW = 2 * _N

_OFFS = {'h': ((0, 1), (0, 2)), 'd': ((1, 1), (2, 2)),
         'b': ((1, 2), (2, 1)), 'v': ((1, 0), (2, 0))}


def _rot_disp(dy, dx, r):
    return [(dy, dx), (dx, -dy), (-dy, -dx), (-dx, dy)][r]


def _out_perm(u, v, r):
    return [(u, v), (v, 1 - u), (1 - u, 1 - v), (1 - v, u)][r]


def _branches():
    out = []
    for ki, k in enumerate(['h', 'd', 'b', 'v']):
        (o1, o2) = _OFFS[k]
        for r in range(4):
            d1 = _rot_disp(o1[0], o1[1], r)
            d2 = _rot_disp(o2[0], o2[1], r)
            perm = [0] * 4
            for u in (0, 1):
                for v in (0, 1):
                    up, vp = _out_perm(u, v, r)
                    perm[u * 2 + v] = up * 2 + vp
            out.append((ki, d1, d2, tuple(perm)))
    return out


_BRANCHES = _branches()
_DYS = (-2, -1, 0, 1, 2)
_DXS = (-2, -1, 0, 1, 2)


def _body(img_ref, tabs_ref, out_ref, *scratch):
    tab_refs = scratch[0:16]
    inbuf = scratch[16]
    outbuf = scratch[17]

    cid = lax.axis_index("c")
    sid = lax.axis_index("s")
    wid = sid * 2 + cid
    row0 = wid * _RPW
    ws = jnp.maximum(jnp.minimum(row0 - 2, _N - _WIN), 0)

    for i in range(16):
        pltpu.sync_copy(tabs_ref.at[i], tab_refs[i])

    iota = lax.iota(jnp.int32, 16)

    for t in range(_PLANES):
        pltpu.sync_copy(img_ref.at[pl.ds(t * _N * _N + ws * _N, _WIN * _N)],
                        inbuf)

        def row_body(i, carry):
            y = row0 + i
            rbs = []
            for dy in _DYS:
                yy = jnp.maximum(jnp.minimum(y + dy, _N - 1), 0)
                rbs.append((yy - ws) * _N)

            def grp_body(g, c2):
                x = g * 16
                cvs = []
                for dx in _DXS:
                    cv = jnp.maximum(jnp.minimum(iota + (x + dx), _N - 1), 0)
                    cvs.append(cv)
                nb = {}
                for dyi, dy in enumerate(_DYS):
                    for dxi, dx in enumerate(_DXS):
                        nb[(dy, dx)] = plsc.load_gather(
                            inbuf, [cvs[dxi] + rbs[dyi]])
                a289 = nb[(0, 0)] * (_L * _L)
                accs = [jnp.zeros((16,), jnp.float32) for _ in range(4)]
                for (ki, d1, d2, perm) in _BRANCHES:
                    idx = a289 + nb[d1] * _L + nb[d2]
                    for uv in range(4):
                        w = plsc.load_gather(tab_refs[ki * 4 + uv], [idx])
                        accs[perm[uv]] = accs[perm[uv]] + w
                stb = iota * 2 + (i * (2 * _OW) + x * 2)
                plsc.store_scatter(outbuf, [stb], accs[0])
                plsc.store_scatter(outbuf, [stb + 1], accs[1])
                plsc.store_scatter(outbuf, [stb + _OW], accs[2])
                plsc.store_scatter(outbuf, [stb + _OW + 1], accs[3])
                return c2

            lax.fori_loop(0, _GROUPS, grp_body, 0)
            return carry

        lax.fori_loop(0, _RPW, row_body, 0)
        pltpu.sync_copy(
            outbuf,
            out_ref.at[pl.ds(t * _OW * _OW + row0 * (2 * _OW),
                             _RPW * 2 * _OW)])


def kernel(img_lr, h_weight, d_weight, b_weight, v_weight):
    img = img_lr.astype(jnp.int32).reshape(_PLANES * _N * _N)
    cols = []
    for w in (h_weight, d_weight, b_weight, v_weight):
        wf = (w * 0.25).reshape(_L ** 3, 4)
        for uv in range(4):
            cols.append(jnp.pad(wf[:, uv], (0, _TAB - _L ** 3)))
    tabs = jnp.stack(cols)

    mesh = plsc.VectorSubcoreMesh(core_axis_name="c", subcore_axis_name="s")
    scratch = [pltpu.VMEM((_TAB,), jnp.float32) for _ in range(16)]
    scratch.append(pltpu.VMEM((_WIN * _N,), jnp.int32))
    scratch.append(pltpu.VMEM((_RPW * 2 * _OW,), jnp.float32))

    out = pl.kernel(
        _body,
        out_type=jax.ShapeDtypeStruct((_PLANES * _OW * _OW,), jnp.float32),
        mesh=mesh,
        scratch_types=scratch,
        compiler_params=pltpu.CompilerParams(needs_layout_passes=False),
    )(img, tabs)
    return out.reshape(2, 3, _OW, _OW)


# speedup vs baseline: 890.9157x; 1.2414x over previous
import jax
import jax.numpy as jnp
from jax import lax
from jax.experimental import pallas as pl
from jax.experimental.pallas import tpu as pltpu
from jax.experimental.pallas import tpu_sc as plsc

_L = 17
_N = 384
_PLANES = 6
_NW = 32
_RPW = _N // _NW
_WIN = _RPW + 4
_TAB = 4920
_GROUPS = _N // 16
_OW = 2 * _N

_OFFS = {'h': ((0, 1), (0, 2)), 'd': ((1, 1), (2, 2)),
         'b': ((1, 2), (2, 1)), 'v': ((1, 0), (2, 0))}


def _rot_disp(dy, dx, r):
    return [(dy, dx), (dx, -dy), (-dy, -dx), (-dx, dy)][r]


def _out_perm(u, v, r):
    return [(u, v), (v, 1 - u), (1 - u, 1 - v), (1 - v, u)][r]


def _idx_groups():
    groups = {}
    order = []
    for ki, k in enumerate(['h', 'd', 'b', 'v']):
        (o1, o2) = _OFFS[k]
        for r in range(4):
            d1 = _rot_disp(o1[0], o1[1], r)
            d2 = _rot_disp(o2[0], o2[1], r)
            perm = [0] * 4
            for u in (0, 1):
                for v in (0, 1):
                    up, vp = _out_perm(u, v, r)
                    perm[u * 2 + v] = up * 2 + vp
            key = (d1, d2)
            if key not in groups:
                groups[key] = []
                order.append(key)
            groups[key].append((ki, tuple(perm)))
    return [(d1, d2, groups[(d1, d2)]) for (d1, d2) in order]


_IDX_GROUPS = _idx_groups()
_DYS = (-2, -1, 0, 1, 2)
_DXS = (-2, -1, 0, 1, 2)
_HI_MASK = -65536


def _body(img_ref, tabs_ref, out_ref, *scratch):
    tab_refs = scratch[0:8]
    inbuf = scratch[8]
    outbuf = scratch[9]

    cid = lax.axis_index("c")
    sid = lax.axis_index("s")
    wid = sid * 2 + cid
    row0 = wid * _RPW
    ws = jnp.maximum(jnp.minimum(row0 - 2, _N - _WIN), 0)

    for i in range(8):
        pltpu.sync_copy(tabs_ref.at[i], tab_refs[i])

    iota = lax.iota(jnp.int32, 16)
    iota2 = iota * 2

    for t in range(_PLANES):
        pltpu.sync_copy(img_ref.at[pl.ds(t * _N * _N + ws * _N, _WIN * _N)],
                        inbuf)

        def row_body(i, carry):
            y = row0 + i
            rbs = []
            for dy in _DYS:
                yy = jnp.maximum(jnp.minimum(y + dy, _N - 1), 0)
                rbs.append((yy - ws) * _N)

            def grp_body(g, c2):
                x = g * 16
                cvs = []
                for dx in _DXS:
                    cv = jnp.maximum(jnp.minimum(iota + (x + dx), _N - 1), 0)
                    cvs.append(cv)
                nb = {}
                for dyi, dy in enumerate(_DYS):
                    for dxi, dx in enumerate(_DXS):
                        nb[(dy, dx)] = plsc.load_gather(
                            inbuf, [cvs[dxi] + rbs[dyi]])
                a289 = nb[(0, 0)] * (_L * _L)
                accs = [jnp.zeros((16,), jnp.float32) for _ in range(4)]
                for (d1, d2, members) in _IDX_GROUPS:
                    idx = a289 + nb[d1] * _L + nb[d2]
                    for (ki, perm) in members:
                        pk_t = plsc.load_gather(tab_refs[ki * 2], [idx])
                        pk_b = plsc.load_gather(tab_refs[ki * 2 + 1], [idx])
                        w00 = plsc.bitcast(lax.shift_left(pk_t, 16),
                                           jnp.float32)
                        w01 = plsc.bitcast(pk_t & _HI_MASK, jnp.float32)
                        w10 = plsc.bitcast(lax.shift_left(pk_b, 16),
                                           jnp.float32)
                        w11 = plsc.bitcast(pk_b & _HI_MASK, jnp.float32)
                        accs[perm[0]] = accs[perm[0]] + w00
                        accs[perm[1]] = accs[perm[1]] + w01
                        accs[perm[2]] = accs[perm[2]] + w10
                        accs[perm[3]] = accs[perm[3]] + w11
                stb = iota2 + (i * (2 * _OW) + x * 2)
                plsc.store_scatter(outbuf, [stb], accs[0])
                plsc.store_scatter(outbuf, [stb + 1], accs[1])
                plsc.store_scatter(outbuf, [stb + _OW], accs[2])
                plsc.store_scatter(outbuf, [stb + _OW + 1], accs[3])
                return c2

            lax.fori_loop(0, _GROUPS, grp_body, 0)
            return carry

        lax.fori_loop(0, _RPW, row_body, 0)
        pltpu.sync_copy(
            outbuf,
            out_ref.at[pl.ds(t * _OW * _OW + row0 * (2 * _OW),
                             _RPW * 2 * _OW)])


def kernel(img_lr, h_weight, d_weight, b_weight, v_weight):
    img = img_lr.astype(jnp.int32).reshape(_PLANES * _N * _N)

    rows = []
    for w in (h_weight, d_weight, b_weight, v_weight):
        wf = (w * 0.25).reshape(_L ** 3, 4)
        bits = lax.bitcast_convert_type(
            wf.astype(jnp.bfloat16), jnp.uint16).astype(jnp.uint32)
        top = lax.bitcast_convert_type(
            (bits[:, 1] << 16) | bits[:, 0], jnp.int32)
        bot = lax.bitcast_convert_type(
            (bits[:, 3] << 16) | bits[:, 2], jnp.int32)
        rows.append(jnp.pad(top, (0, _TAB - _L ** 3)))
        rows.append(jnp.pad(bot, (0, _TAB - _L ** 3)))
    tabs = jnp.stack(rows)

    mesh = plsc.VectorSubcoreMesh(core_axis_name="c", subcore_axis_name="s")
    scratch = [pltpu.VMEM((_TAB,), jnp.int32) for _ in range(8)]
    scratch.append(pltpu.VMEM((_WIN * _N,), jnp.int32))
    scratch.append(pltpu.VMEM((_RPW * 2 * _OW,), jnp.float32))

    out = pl.kernel(
        _body,
        out_type=jax.ShapeDtypeStruct((_PLANES * _OW * _OW,), jnp.float32),
        mesh=mesh,
        scratch_types=scratch,
        compiler_params=pltpu.CompilerParams(needs_layout_passes=False),
    )(img, tabs)
    return out.reshape(2, 3, _OW, _OW)


# speedup vs baseline: 981.8168x; 1.1020x over previous
import jax
import jax.numpy as jnp
from jax import lax
from jax.experimental import pallas as pl
from jax.experimental.pallas import tpu as pltpu
from jax.experimental.pallas import tpu_sc as plsc

_L = 17
_N = 384
_PLANES = 6
_NW = 32
_RPW = _N // _NW
_WIN = _RPW + 4
_TAB = 4920
_GROUPS = _N // 16
_OW = 2 * _N

_OFFS = {'h': ((0, 1), (0, 2)), 'd': ((1, 1), (2, 2)),
         'b': ((1, 2), (2, 1)), 'v': ((1, 0), (2, 0))}


def _rot_disp(dy, dx, r):
    return [(dy, dx), (dx, -dy), (-dy, -dx), (-dx, dy)][r]


def _out_perm(u, v, r):
    return [(u, v), (v, 1 - u), (1 - u, 1 - v), (1 - v, u)][r]


def _idx_groups():
    groups = {}
    order = []
    for ki, k in enumerate(['h', 'd', 'b', 'v']):
        (o1, o2) = _OFFS[k]
        for r in range(4):
            d1 = _rot_disp(o1[0], o1[1], r)
            d2 = _rot_disp(o2[0], o2[1], r)
            perm = [0] * 4
            for u in (0, 1):
                for v in (0, 1):
                    up, vp = _out_perm(u, v, r)
                    perm[u * 2 + v] = up * 2 + vp
            key = (d1, d2)
            if key not in groups:
                groups[key] = []
                order.append(key)
            groups[key].append((ki, tuple(perm)))
    return [(d1, d2, groups[(d1, d2)]) for (d1, d2) in order]


_IDX_GROUPS = _idx_groups()
_DYS = (-2, -1, 0, 1, 2)
_DXS = (-2, -1, 0, 1, 2)
_HI_MASK = -65536


def _body(img_ref, tabs_ref, out_ref, *scratch):
    tab_refs = scratch[0:8]
    inbuf = scratch[8]
    outbuf = scratch[9]

    cid = lax.axis_index("c")
    sid = lax.axis_index("s")
    wid = sid * 2 + cid
    row0 = wid * _RPW
    ws = jnp.maximum(jnp.minimum(row0 - 2, _N - _WIN), 0)

    for i in range(8):
        pltpu.sync_copy(tabs_ref.at[i], tab_refs[i])

    iota = lax.iota(jnp.int32, 16)
    iota2 = iota * 2

    for t in range(_PLANES):
        pltpu.sync_copy(img_ref.at[pl.ds(t * _N * _N + ws * _N, _WIN * _N)],
                        inbuf)

        def pair_body(p, carry):
            y0 = row0 + 2 * p
            rbs = []
            for j in range(6):
                yy = jnp.maximum(jnp.minimum(y0 - 2 + j, _N - 1), 0)
                rbs.append((yy - ws) * _N)

            def grp_body(g, c2):
                x = g * 16
                cvs = {}
                for dx in _DXS:
                    cvs[dx] = jnp.maximum(
                        jnp.minimum(iota + (x + dx), _N - 1), 0)
                loads = {}
                for j in range(6):
                    for dx in _DXS:
                        loads[(j, dx)] = plsc.load_gather(
                            inbuf, [cvs[dx] + rbs[j]])
                for r in (0, 1):
                    nb = {(dy, dx): loads[(dy + 2 + r, dx)]
                          for dy in _DYS for dx in _DXS}
                    a289 = nb[(0, 0)] * (_L * _L)
                    accs = [jnp.zeros((16,), jnp.float32) for _ in range(4)]
                    for (d1, d2, members) in _IDX_GROUPS:
                        idx = a289 + nb[d1] * _L + nb[d2]
                        for (ki, perm) in members:
                            pk_t = plsc.load_gather(tab_refs[ki * 2], [idx])
                            pk_b = plsc.load_gather(tab_refs[ki * 2 + 1],
                                                    [idx])
                            w00 = plsc.bitcast(lax.shift_left(pk_t, 16),
                                               jnp.float32)
                            w01 = plsc.bitcast(pk_t, jnp.float32)
                            w10 = plsc.bitcast(lax.shift_left(pk_b, 16),
                                               jnp.float32)
                            w11 = plsc.bitcast(pk_b, jnp.float32)
                            accs[perm[0]] = accs[perm[0]] + w00
                            accs[perm[1]] = accs[perm[1]] + w01
                            accs[perm[2]] = accs[perm[2]] + w10
                            accs[perm[3]] = accs[perm[3]] + w11
                    stb = iota2 + ((2 * p + r) * (2 * _OW) + x * 2)
                    plsc.store_scatter(outbuf, [stb], accs[0])
                    plsc.store_scatter(outbuf, [stb + 1], accs[1])
                    plsc.store_scatter(outbuf, [stb + _OW], accs[2])
                    plsc.store_scatter(outbuf, [stb + _OW + 1], accs[3])
                return c2

            lax.fori_loop(0, _GROUPS, grp_body, 0)
            return carry

        lax.fori_loop(0, _RPW // 2, pair_body, 0)
        pltpu.sync_copy(
            outbuf,
            out_ref.at[pl.ds(t * _OW * _OW + row0 * (2 * _OW),
                             _RPW * 2 * _OW)])


def kernel(img_lr, h_weight, d_weight, b_weight, v_weight):
    img = img_lr.astype(jnp.int32).reshape(_PLANES * _N * _N)

    rows = []
    for w in (h_weight, d_weight, b_weight, v_weight):
        wf = (w * 0.25).reshape(_L ** 3, 4)
        bits = lax.bitcast_convert_type(
            wf.astype(jnp.bfloat16), jnp.uint16).astype(jnp.uint32)
        top = lax.bitcast_convert_type(
            (bits[:, 1] << 16) | bits[:, 0], jnp.int32)
        bot = lax.bitcast_convert_type(
            (bits[:, 3] << 16) | bits[:, 2], jnp.int32)
        rows.append(jnp.pad(top, (0, _TAB - _L ** 3)))
        rows.append(jnp.pad(bot, (0, _TAB - _L ** 3)))
    tabs = jnp.stack(rows)

    mesh = plsc.VectorSubcoreMesh(core_axis_name="c", subcore_axis_name="s")
    scratch = [pltpu.VMEM((_TAB,), jnp.int32) for _ in range(8)]
    scratch.append(pltpu.VMEM((_WIN * _N,), jnp.int32))
    scratch.append(pltpu.VMEM((_RPW * 2 * _OW,), jnp.float32))

    out = pl.kernel(
        _body,
        out_type=jax.ShapeDtypeStruct((_PLANES * _OW * _OW,), jnp.float32),
        mesh=mesh,
        scratch_types=scratch,
        compiler_params=pltpu.CompilerParams(needs_layout_passes=False),
    )(img, tabs)
    return out.reshape(2, 3, _OW, _OW)
